# baseline (speedup 1.0000x reference)
import functools

import jax
import jax.numpy as jnp
from jax import lax
from jax.experimental import pallas as pl
from jax.experimental.pallas import tpu as pltpu
from jax.experimental.pallas import tpu_sc as plsc

N = 10000
E = 320000
D_IN = 128
D_HID = 128
D_HALF = D_HID // 2
D_OUT = 40

NC = 2
NS = 16
NW = NC * NS
K = 100
E_PER_TILE = E // NS
CH1 = E_PER_TILE // K
E_PER_W = E // NW
CH2 = E_PER_W // K
N_PAD = 10240
ROWS_PER_TILE = N_PAD // NS
NBUF = 4

_f32 = jnp.float32


def _sc_mesh():
    return plsc.VectorSubcoreMesh(core_axis_name="c", subcore_axis_name="s",
                                  num_cores=NC, num_subcores=NS)


def _pipelined_agg(ch, gather_start, gather_wait, scat_start, scat_wait):
    for u in range(NBUF - 1):
        gather_start(u, u)

    def step(j, b):
        gather_wait(j, b)
        scat_start(j, b)
        if not (isinstance(j, int) and j == 0):
            scat_wait(None, (b - 1) % NBUF)
        gather_start(jnp.minimum(j + NBUF - 1, ch - 1), (b + NBUF - 1) % NBUF)

    for j in range(NBUF):
        step(j, j % NBUF)

    def body(t, carry):
        for u in range(NBUF):
            step(t * NBUF + u, u)
        return carry

    lax.fori_loop(1, ch // NBUF, body, 0)

    scat_wait(None, (ch - 1) % NBUF)
    for u in range(NBUF - 1):
        gather_wait(0, u)


@functools.partial(
    pl.kernel,
    out_type=[
        jax.ShapeDtypeStruct((NC, N_PAD, D_HALF), _f32),
        jax.ShapeDtypeStruct((NC, N_PAD, 8), _f32),
    ],
    mesh=_sc_mesh(),
    compiler_params=pltpu.CompilerParams(use_tc_tiling_on_sc=False),
    scratch_types=[
        pltpu.VMEM((CH1, K), jnp.int32),
        pltpu.VMEM((CH1, K), jnp.int32),
        [pltpu.VMEM((K, D_HALF), _f32)] * NBUF,
        pltpu.VMEM((K, 8), _f32),
        pltpu.VMEM_SHARED((N_PAD, D_HALF), _f32),
        pltpu.VMEM_SHARED((N_PAD, 8), _f32),
        [pltpu.SemaphoreType.DMA] * NBUF,
        [pltpu.SemaphoreType.DMA] * NBUF,
    ],
)
def _sc_agg1(h_hbm, src_hbm, dst_hbm, z_feat_hbm, z_deg_hbm, ones_hbm,
             out_hbm, deg_out_hbm,
             sidx, didx, rows, ones_v, acc, dacc, gsem, ssem):
    c = lax.axis_index("c")
    s = lax.axis_index("s")

    pltpu.sync_copy(z_feat_hbm, acc.at[pl.ds(s * ROWS_PER_TILE, ROWS_PER_TILE)])
    pltpu.sync_copy(z_deg_hbm, dacc.at[pl.ds(s * ROWS_PER_TILE, ROWS_PER_TILE)])
    pltpu.sync_copy(src_hbm.at[c, s], sidx)
    pltpu.sync_copy(dst_hbm.at[s], didx)
    pltpu.sync_copy(ones_hbm, ones_v)
    plsc.subcore_barrier()

    def gather_start(j, b):
        pltpu.async_copy(h_hbm.at[sidx.at[j]], rows[b], gsem[b])

    def gather_wait(j, b):
        pltpu.make_async_copy(h_hbm.at[sidx.at[0]], rows[b],
                              gsem[b]).wait()

    def scat_start(j, b):
        pltpu.async_copy(rows[b], acc.at[didx.at[j]], ssem[b], add=True)
        pltpu.async_copy(ones_v, dacc.at[didx.at[j]], ssem[b], add=True)

    def scat_wait(_, b):
        pltpu.make_async_copy(rows[b], acc.at[didx.at[0]], ssem[b]).wait()
        pltpu.make_async_copy(ones_v, dacc.at[didx.at[0]], ssem[b]).wait()

    _pipelined_agg(CH1, gather_start, gather_wait, scat_start, scat_wait)
    plsc.subcore_barrier()

    pltpu.sync_copy(acc.at[pl.ds(s * ROWS_PER_TILE, ROWS_PER_TILE)],
                    out_hbm.at[c, pl.ds(s * ROWS_PER_TILE, ROWS_PER_TILE)])
    pltpu.sync_copy(dacc.at[pl.ds(s * ROWS_PER_TILE, ROWS_PER_TILE)],
                    deg_out_hbm.at[c, pl.ds(s * ROWS_PER_TILE, ROWS_PER_TILE)])


@functools.partial(
    pl.kernel,
    out_type=jax.ShapeDtypeStruct((NC, N_PAD, D_OUT), _f32),
    mesh=_sc_mesh(),
    compiler_params=pltpu.CompilerParams(use_tc_tiling_on_sc=False),
    scratch_types=[
        pltpu.VMEM((CH2, K), jnp.int32),
        pltpu.VMEM((CH2, K), jnp.int32),
        [pltpu.VMEM((K, D_OUT), _f32)] * NBUF,
        pltpu.VMEM_SHARED((N_PAD, D_OUT), _f32),
        [pltpu.SemaphoreType.DMA] * NBUF,
        [pltpu.SemaphoreType.DMA] * NBUF,
    ],
)
def _sc_agg2(h_hbm, src_hbm, dst_hbm, z_feat_hbm,
             out_hbm,
             sidx, didx, rows, acc, gsem, ssem):
    c = lax.axis_index("c")
    s = lax.axis_index("s")

    pltpu.sync_copy(z_feat_hbm, acc.at[pl.ds(s * ROWS_PER_TILE, ROWS_PER_TILE)])
    pltpu.sync_copy(src_hbm.at[s, pl.ds(c * CH2, CH2)], sidx)
    pltpu.sync_copy(dst_hbm.at[s, pl.ds(c * CH2, CH2)], didx)
    plsc.subcore_barrier()

    def gather_start(j, b):
        pltpu.async_copy(h_hbm.at[sidx.at[j]], rows[b], gsem[b])

    def gather_wait(j, b):
        pltpu.make_async_copy(h_hbm.at[sidx.at[0]], rows[b], gsem[b]).wait()

    def scat_start(j, b):
        pltpu.async_copy(rows[b], acc.at[didx.at[j]], ssem[b], add=True)

    def scat_wait(_, b):
        pltpu.make_async_copy(rows[b], acc.at[didx.at[0]], ssem[b]).wait()

    _pipelined_agg(CH2, gather_start, gather_wait, scat_start, scat_wait)
    plsc.subcore_barrier()

    pltpu.sync_copy(acc.at[pl.ds(s * ROWS_PER_TILE, ROWS_PER_TILE)],
                    out_hbm.at[c, pl.ds(s * ROWS_PER_TILE, ROWS_PER_TILE)])


_BLK = 1000


def _mid_body(a_ref, d_ref, w1_ref, b1_ref, w2_ref, o_ref):
    d = d_ref[...][0, :, 0:1]
    deg_inv = 1.0 / jnp.maximum(d, 1.0)
    a = a_ref[...]
    ax = jnp.concatenate([a[0], a[1]], axis=1)
    h1 = jnp.dot(ax, w1_ref[...], preferred_element_type=_f32)
    x1 = jnp.maximum(h1 * deg_inv + b1_ref[...], 0.0)
    o_ref[...] = jnp.dot(x1, w2_ref[...], preferred_element_type=_f32)


def _tc_mid(a, d, w1, b1, w2):
    return pl.pallas_call(
        _mid_body,
        grid=(N // _BLK,),
        in_specs=[
            pl.BlockSpec((2, _BLK, D_HALF), lambda i: (0, i, 0)),
            pl.BlockSpec((1, _BLK, 8), lambda i: (0, i, 0)),
            pl.BlockSpec((D_IN, D_HID), lambda i: (0, 0)),
            pl.BlockSpec((1, D_HID), lambda i: (0, 0)),
            pl.BlockSpec((D_HID, D_OUT), lambda i: (0, 0)),
        ],
        out_specs=pl.BlockSpec((_BLK, D_OUT), lambda i: (i, 0)),
        out_shape=jax.ShapeDtypeStruct((N, D_OUT), _f32),
    )(a, d, w1, b1, w2)


def _final_body(g_ref, d_ref, b2_ref, o_ref):
    d = d_ref[...][0, :, 0:1]
    deg_inv = 1.0 / jnp.maximum(d, 1.0)
    g = g_ref[...]
    x = (g[0] + g[1]) * deg_inv + b2_ref[...]
    m = jnp.max(x, axis=1, keepdims=True)
    lse = m + jnp.log(jnp.sum(jnp.exp(x - m), axis=1, keepdims=True))
    o_ref[...] = x - lse


def _tc_final(g, d, b2):
    return pl.pallas_call(
        _final_body,
        grid=(N // _BLK,),
        in_specs=[
            pl.BlockSpec((2, _BLK, D_OUT), lambda i: (0, i, 0)),
            pl.BlockSpec((1, _BLK, 8), lambda i: (0, i, 0)),
            pl.BlockSpec((1, D_OUT), lambda i: (0, 0)),
        ],
        out_specs=pl.BlockSpec((_BLK, D_OUT), lambda i: (i, 0)),
        out_shape=jax.ShapeDtypeStruct((N, D_OUT), _f32),
    )(g, d, b2)


def kernel(features, edge_index, W1, b1, W2, b2):
    src_t = edge_index[0].reshape(NS, CH1, K)
    dst_t = edge_index[1].reshape(NS, CH1, K)
    src2 = jnp.stack([2 * src_t, 2 * src_t + 1])
    feat2 = features.reshape(2 * N, D_HALF)

    z_half = jnp.zeros((ROWS_PER_TILE, D_HALF), _f32)
    z_deg = jnp.zeros((ROWS_PER_TILE, 8), _f32)
    z_out = jnp.zeros((ROWS_PER_TILE, D_OUT), _f32)
    ones_rows = jnp.ones((K, 8), _f32)

    agg1, deg16 = _sc_agg1(feat2, src2, dst_t, z_half, z_deg, ones_rows)

    h2 = _tc_mid(agg1, deg16, W1, b1.reshape(1, D_HID), W2)

    agg2 = _sc_agg2(h2, src_t, dst_t, z_out)
    out = _tc_final(agg2, deg16, b2.reshape(1, D_OUT))
    return out


# speedup vs baseline: 1.0356x; 1.0356x over previous
import functools

import jax
import jax.numpy as jnp
from jax import lax
from jax.experimental import pallas as pl
from jax.experimental.pallas import tpu as pltpu
from jax.experimental.pallas import tpu_sc as plsc

N = 10000
E = 320000
D_IN = 128
D_HID = 128
D_HALF = D_HID // 2
D_OUT = 40

NC = 2
NS = 16
NW = NC * NS
K = 125
E_PER_TILE = E // NS
CH1 = E_PER_TILE // K
E_PER_W = E // NW
CH2 = E_PER_W // K
N_PAD = 10240
ROWS_PER_TILE = N_PAD // NS
NBUF = 4

_f32 = jnp.float32


def _sc_mesh():
    return plsc.VectorSubcoreMesh(core_axis_name="c", subcore_axis_name="s",
                                  num_cores=NC, num_subcores=NS)


def _pipelined_agg(ch, gather_start, gather_wait, scat_start, scat_wait):
    for u in range(NBUF - 1):
        gather_start(u, u)

    def step(j, b):
        gather_wait(j, b)
        scat_start(j, b)
        if not (isinstance(j, int) and j == 0):
            scat_wait(None, (b - 1) % NBUF)
        gather_start(jnp.minimum(j + NBUF - 1, ch - 1), (b + NBUF - 1) % NBUF)

    for j in range(NBUF):
        step(j, j % NBUF)

    def body(t, carry):
        for u in range(NBUF):
            step(t * NBUF + u, u)
        return carry

    lax.fori_loop(1, ch // NBUF, body, 0)

    scat_wait(None, (ch - 1) % NBUF)
    for u in range(NBUF - 1):
        gather_wait(0, u)


@functools.partial(
    pl.kernel,
    out_type=[
        jax.ShapeDtypeStruct((NC, N_PAD, D_HALF), _f32),
        jax.ShapeDtypeStruct((NC, N_PAD, 8), _f32),
    ],
    mesh=_sc_mesh(),
    compiler_params=pltpu.CompilerParams(use_tc_tiling_on_sc=False),
    scratch_types=[
        pltpu.VMEM((CH1, K), jnp.int32),
        pltpu.VMEM((CH1, K), jnp.int32),
        [pltpu.VMEM((K, D_HALF), _f32)] * NBUF,
        pltpu.VMEM((K, 8), _f32),
        pltpu.VMEM_SHARED((N_PAD, D_HALF), _f32),
        pltpu.VMEM_SHARED((N_PAD, 8), _f32),
        [pltpu.SemaphoreType.DMA] * NBUF,
        [pltpu.SemaphoreType.DMA] * NBUF,
    ],
)
def _sc_agg1(h_hbm, src_hbm, dst_hbm, z_feat_hbm, z_deg_hbm, ones_hbm,
             out_hbm, deg_out_hbm,
             sidx, didx, rows, ones_v, acc, dacc, gsem, ssem):
    c = lax.axis_index("c")
    s = lax.axis_index("s")

    pltpu.sync_copy(z_feat_hbm, acc.at[pl.ds(s * ROWS_PER_TILE, ROWS_PER_TILE)])
    pltpu.sync_copy(z_deg_hbm, dacc.at[pl.ds(s * ROWS_PER_TILE, ROWS_PER_TILE)])
    pltpu.sync_copy(src_hbm.at[c, s], sidx)
    pltpu.sync_copy(dst_hbm.at[s], didx)
    pltpu.sync_copy(ones_hbm, ones_v)
    plsc.subcore_barrier()

    def gather_start(j, b):
        pltpu.async_copy(h_hbm.at[sidx.at[j]], rows[b], gsem[b])

    def gather_wait(j, b):
        pltpu.make_async_copy(h_hbm.at[sidx.at[0]], rows[b],
                              gsem[b]).wait()

    def scat_start(j, b):
        pltpu.async_copy(rows[b], acc.at[didx.at[j]], ssem[b], add=True)
        pltpu.async_copy(ones_v, dacc.at[didx.at[j]], ssem[b], add=True)

    def scat_wait(_, b):
        pltpu.make_async_copy(rows[b], acc.at[didx.at[0]], ssem[b]).wait()
        pltpu.make_async_copy(ones_v, dacc.at[didx.at[0]], ssem[b]).wait()

    _pipelined_agg(CH1, gather_start, gather_wait, scat_start, scat_wait)
    plsc.subcore_barrier()

    pltpu.sync_copy(acc.at[pl.ds(s * ROWS_PER_TILE, ROWS_PER_TILE)],
                    out_hbm.at[c, pl.ds(s * ROWS_PER_TILE, ROWS_PER_TILE)])
    pltpu.sync_copy(dacc.at[pl.ds(s * ROWS_PER_TILE, ROWS_PER_TILE)],
                    deg_out_hbm.at[c, pl.ds(s * ROWS_PER_TILE, ROWS_PER_TILE)])


@functools.partial(
    pl.kernel,
    out_type=jax.ShapeDtypeStruct((NC, N_PAD, D_OUT), _f32),
    mesh=_sc_mesh(),
    compiler_params=pltpu.CompilerParams(use_tc_tiling_on_sc=False),
    scratch_types=[
        pltpu.VMEM((CH2, K), jnp.int32),
        pltpu.VMEM((CH2, K), jnp.int32),
        [pltpu.VMEM((K, D_OUT), _f32)] * NBUF,
        pltpu.VMEM_SHARED((N_PAD, D_OUT), _f32),
        [pltpu.SemaphoreType.DMA] * NBUF,
        [pltpu.SemaphoreType.DMA] * NBUF,
    ],
)
def _sc_agg2(h_hbm, src_hbm, dst_hbm, z_feat_hbm,
             out_hbm,
             sidx, didx, rows, acc, gsem, ssem):
    c = lax.axis_index("c")
    s = lax.axis_index("s")

    pltpu.sync_copy(z_feat_hbm, acc.at[pl.ds(s * ROWS_PER_TILE, ROWS_PER_TILE)])
    pltpu.sync_copy(src_hbm.at[s, pl.ds(c * CH2, CH2)], sidx)
    pltpu.sync_copy(dst_hbm.at[s, pl.ds(c * CH2, CH2)], didx)
    plsc.subcore_barrier()

    def gather_start(j, b):
        pltpu.async_copy(h_hbm.at[sidx.at[j]], rows[b], gsem[b])

    def gather_wait(j, b):
        pltpu.make_async_copy(h_hbm.at[sidx.at[0]], rows[b], gsem[b]).wait()

    def scat_start(j, b):
        pltpu.async_copy(rows[b], acc.at[didx.at[j]], ssem[b], add=True)

    def scat_wait(_, b):
        pltpu.make_async_copy(rows[b], acc.at[didx.at[0]], ssem[b]).wait()

    _pipelined_agg(CH2, gather_start, gather_wait, scat_start, scat_wait)
    plsc.subcore_barrier()

    pltpu.sync_copy(acc.at[pl.ds(s * ROWS_PER_TILE, ROWS_PER_TILE)],
                    out_hbm.at[c, pl.ds(s * ROWS_PER_TILE, ROWS_PER_TILE)])


_BLK = 1000


def _mid_body(a_ref, d_ref, w1_ref, b1_ref, w2_ref, o_ref):
    d = d_ref[...][0, :, 0:1]
    deg_inv = 1.0 / jnp.maximum(d, 1.0)
    a = a_ref[...]
    ax = jnp.concatenate([a[0], a[1]], axis=1)
    h1 = jnp.dot(ax, w1_ref[...], preferred_element_type=_f32)
    x1 = jnp.maximum(h1 * deg_inv + b1_ref[...], 0.0)
    o_ref[...] = jnp.dot(x1, w2_ref[...], preferred_element_type=_f32)


def _tc_mid(a, d, w1, b1, w2):
    return pl.pallas_call(
        _mid_body,
        grid=(N // _BLK,),
        in_specs=[
            pl.BlockSpec((2, _BLK, D_HALF), lambda i: (0, i, 0)),
            pl.BlockSpec((1, _BLK, 8), lambda i: (0, i, 0)),
            pl.BlockSpec((D_IN, D_HID), lambda i: (0, 0)),
            pl.BlockSpec((1, D_HID), lambda i: (0, 0)),
            pl.BlockSpec((D_HID, D_OUT), lambda i: (0, 0)),
        ],
        out_specs=pl.BlockSpec((_BLK, D_OUT), lambda i: (i, 0)),
        out_shape=jax.ShapeDtypeStruct((N, D_OUT), _f32),
    )(a, d, w1, b1, w2)


def _final_body(g_ref, d_ref, b2_ref, o_ref):
    d = d_ref[...][0, :, 0:1]
    deg_inv = 1.0 / jnp.maximum(d, 1.0)
    g = g_ref[...]
    x = (g[0] + g[1]) * deg_inv + b2_ref[...]
    m = jnp.max(x, axis=1, keepdims=True)
    lse = m + jnp.log(jnp.sum(jnp.exp(x - m), axis=1, keepdims=True))
    o_ref[...] = x - lse


def _tc_final(g, d, b2):
    return pl.pallas_call(
        _final_body,
        grid=(N // _BLK,),
        in_specs=[
            pl.BlockSpec((2, _BLK, D_OUT), lambda i: (0, i, 0)),
            pl.BlockSpec((1, _BLK, 8), lambda i: (0, i, 0)),
            pl.BlockSpec((1, D_OUT), lambda i: (0, 0)),
        ],
        out_specs=pl.BlockSpec((_BLK, D_OUT), lambda i: (i, 0)),
        out_shape=jax.ShapeDtypeStruct((N, D_OUT), _f32),
    )(g, d, b2)


def kernel(features, edge_index, W1, b1, W2, b2):
    src_t = edge_index[0].reshape(NS, CH1, K)
    dst_t = edge_index[1].reshape(NS, CH1, K)
    src2 = jnp.stack([2 * src_t, 2 * src_t + 1])
    feat2 = features.reshape(2 * N, D_HALF)

    z_half = jnp.zeros((ROWS_PER_TILE, D_HALF), _f32)
    z_deg = jnp.zeros((ROWS_PER_TILE, 8), _f32)
    z_out = jnp.zeros((ROWS_PER_TILE, D_OUT), _f32)
    ones_rows = jnp.ones((K, 8), _f32)

    agg1, deg16 = _sc_agg1(feat2, src2, dst_t, z_half, z_deg, ones_rows)

    h2 = _tc_mid(agg1, deg16, W1, b1.reshape(1, D_HID), W2)

    agg2 = _sc_agg2(h2, src_t, dst_t, z_out)
    out = _tc_final(agg2, deg16, b2.reshape(1, D_OUT))
    return out


# speedup vs baseline: 1.0700x; 1.0333x over previous
import functools

import jax
import jax.numpy as jnp
from jax import lax
from jax.experimental import pallas as pl
from jax.experimental.pallas import tpu as pltpu
from jax.experimental.pallas import tpu_sc as plsc

N = 10000
E = 320000
D_IN = 128
D_HID = 128
D_HALF = D_HID // 2
D_OUT = 40

NC = 2
NS = 16
NW = NC * NS
K = 125
E_PER_TILE = E // NS
CH1 = E_PER_TILE // K
E_PER_W = E // NW
CH2 = E_PER_W // K
N_PAD = 10240
ROWS_PER_TILE = N_PAD // NS
NBUF = 5

_f32 = jnp.float32


def _sc_mesh():
    return plsc.VectorSubcoreMesh(core_axis_name="c", subcore_axis_name="s",
                                  num_cores=NC, num_subcores=NS)


def _pipelined_agg(ch, gather_start, gather_wait, scat_start, scat_wait):
    for u in range(NBUF - 1):
        gather_start(u, u)

    def step(j, b):
        gather_wait(j, b)
        scat_start(j, b)
        if not (isinstance(j, int) and j == 0):
            scat_wait(None, (b - 1) % NBUF)
        gather_start(jnp.minimum(j + NBUF - 1, ch - 1), (b + NBUF - 1) % NBUF)

    for j in range(NBUF):
        step(j, j % NBUF)

    def body(t, carry):
        for u in range(NBUF):
            step(t * NBUF + u, u)
        return carry

    lax.fori_loop(1, ch // NBUF, body, 0)

    scat_wait(None, (ch - 1) % NBUF)
    for u in range(NBUF - 1):
        gather_wait(0, u)


@functools.partial(
    pl.kernel,
    out_type=[
        jax.ShapeDtypeStruct((NC, N_PAD, D_HALF), _f32),
        jax.ShapeDtypeStruct((NC, N_PAD, 8), _f32),
    ],
    mesh=_sc_mesh(),
    compiler_params=pltpu.CompilerParams(use_tc_tiling_on_sc=False),
    scratch_types=[
        pltpu.VMEM((CH1, K), jnp.int32),
        pltpu.VMEM((CH1, K), jnp.int32),
        [pltpu.VMEM((K, D_HALF), _f32)] * NBUF,
        pltpu.VMEM((K, 8), _f32),
        pltpu.VMEM_SHARED((N_PAD, D_HALF), _f32),
        pltpu.VMEM_SHARED((N_PAD, 8), _f32),
        [pltpu.SemaphoreType.DMA] * NBUF,
        [pltpu.SemaphoreType.DMA] * NBUF,
    ],
)
def _sc_agg1(h_hbm, src_hbm, dst_hbm, z_feat_hbm, z_deg_hbm, ones_hbm,
             out_hbm, deg_out_hbm,
             sidx, didx, rows, ones_v, acc, dacc, gsem, ssem):
    c = lax.axis_index("c")
    s = lax.axis_index("s")

    pltpu.sync_copy(z_feat_hbm, acc.at[pl.ds(s * ROWS_PER_TILE, ROWS_PER_TILE)])
    pltpu.sync_copy(z_deg_hbm, dacc.at[pl.ds(s * ROWS_PER_TILE, ROWS_PER_TILE)])
    pltpu.sync_copy(src_hbm.at[c, s], sidx)
    pltpu.sync_copy(dst_hbm.at[s], didx)
    pltpu.sync_copy(ones_hbm, ones_v)
    plsc.subcore_barrier()

    def gather_start(j, b):
        pltpu.async_copy(h_hbm.at[sidx.at[j]], rows[b], gsem[b])

    def gather_wait(j, b):
        pltpu.make_async_copy(h_hbm.at[sidx.at[0]], rows[b],
                              gsem[b]).wait()

    def scat_start(j, b):
        pltpu.async_copy(rows[b], acc.at[didx.at[j]], ssem[b], add=True)
        pltpu.async_copy(ones_v, dacc.at[didx.at[j]], ssem[b], add=True)

    def scat_wait(_, b):
        pltpu.make_async_copy(rows[b], acc.at[didx.at[0]], ssem[b]).wait()
        pltpu.make_async_copy(ones_v, dacc.at[didx.at[0]], ssem[b]).wait()

    _pipelined_agg(CH1, gather_start, gather_wait, scat_start, scat_wait)
    plsc.subcore_barrier()

    pltpu.sync_copy(acc.at[pl.ds(s * ROWS_PER_TILE, ROWS_PER_TILE)],
                    out_hbm.at[c, pl.ds(s * ROWS_PER_TILE, ROWS_PER_TILE)])
    pltpu.sync_copy(dacc.at[pl.ds(s * ROWS_PER_TILE, ROWS_PER_TILE)],
                    deg_out_hbm.at[c, pl.ds(s * ROWS_PER_TILE, ROWS_PER_TILE)])


@functools.partial(
    pl.kernel,
    out_type=jax.ShapeDtypeStruct((NC, N_PAD, D_OUT), _f32),
    mesh=_sc_mesh(),
    compiler_params=pltpu.CompilerParams(use_tc_tiling_on_sc=False),
    scratch_types=[
        pltpu.VMEM((CH2, K), jnp.int32),
        pltpu.VMEM((CH2, K), jnp.int32),
        [pltpu.VMEM((K, D_OUT), _f32)] * NBUF,
        pltpu.VMEM_SHARED((N_PAD, D_OUT), _f32),
        [pltpu.SemaphoreType.DMA] * NBUF,
        [pltpu.SemaphoreType.DMA] * NBUF,
    ],
)
def _sc_agg2(h_hbm, src_hbm, dst_hbm, z_feat_hbm,
             out_hbm,
             sidx, didx, rows, acc, gsem, ssem):
    c = lax.axis_index("c")
    s = lax.axis_index("s")

    pltpu.sync_copy(z_feat_hbm, acc.at[pl.ds(s * ROWS_PER_TILE, ROWS_PER_TILE)])
    pltpu.sync_copy(src_hbm.at[s, pl.ds(c * CH2, CH2)], sidx)
    pltpu.sync_copy(dst_hbm.at[s, pl.ds(c * CH2, CH2)], didx)
    plsc.subcore_barrier()

    def gather_start(j, b):
        pltpu.async_copy(h_hbm.at[sidx.at[j]], rows[b], gsem[b])

    def gather_wait(j, b):
        pltpu.make_async_copy(h_hbm.at[sidx.at[0]], rows[b], gsem[b]).wait()

    def scat_start(j, b):
        pltpu.async_copy(rows[b], acc.at[didx.at[j]], ssem[b], add=True)

    def scat_wait(_, b):
        pltpu.make_async_copy(rows[b], acc.at[didx.at[0]], ssem[b]).wait()

    _pipelined_agg(CH2, gather_start, gather_wait, scat_start, scat_wait)
    plsc.subcore_barrier()

    pltpu.sync_copy(acc.at[pl.ds(s * ROWS_PER_TILE, ROWS_PER_TILE)],
                    out_hbm.at[c, pl.ds(s * ROWS_PER_TILE, ROWS_PER_TILE)])


_BLK = 1000


def _mid_body(a_ref, d_ref, w1_ref, b1_ref, w2_ref, o_ref):
    d = d_ref[...][0, :, 0:1]
    deg_inv = 1.0 / jnp.maximum(d, 1.0)
    a = a_ref[...]
    ax = jnp.concatenate([a[0], a[1]], axis=1)
    h1 = jnp.dot(ax, w1_ref[...], preferred_element_type=_f32)
    x1 = jnp.maximum(h1 * deg_inv + b1_ref[...], 0.0)
    o_ref[...] = jnp.dot(x1, w2_ref[...], preferred_element_type=_f32)


def _tc_mid(a, d, w1, b1, w2):
    return pl.pallas_call(
        _mid_body,
        grid=(N // _BLK,),
        in_specs=[
            pl.BlockSpec((2, _BLK, D_HALF), lambda i: (0, i, 0)),
            pl.BlockSpec((1, _BLK, 8), lambda i: (0, i, 0)),
            pl.BlockSpec((D_IN, D_HID), lambda i: (0, 0)),
            pl.BlockSpec((1, D_HID), lambda i: (0, 0)),
            pl.BlockSpec((D_HID, D_OUT), lambda i: (0, 0)),
        ],
        out_specs=pl.BlockSpec((_BLK, D_OUT), lambda i: (i, 0)),
        out_shape=jax.ShapeDtypeStruct((N, D_OUT), _f32),
    )(a, d, w1, b1, w2)


def _final_body(g_ref, d_ref, b2_ref, o_ref):
    d = d_ref[...][0, :, 0:1]
    deg_inv = 1.0 / jnp.maximum(d, 1.0)
    g = g_ref[...]
    x = (g[0] + g[1]) * deg_inv + b2_ref[...]
    m = jnp.max(x, axis=1, keepdims=True)
    lse = m + jnp.log(jnp.sum(jnp.exp(x - m), axis=1, keepdims=True))
    o_ref[...] = x - lse


def _tc_final(g, d, b2):
    return pl.pallas_call(
        _final_body,
        grid=(N // _BLK,),
        in_specs=[
            pl.BlockSpec((2, _BLK, D_OUT), lambda i: (0, i, 0)),
            pl.BlockSpec((1, _BLK, 8), lambda i: (0, i, 0)),
            pl.BlockSpec((1, D_OUT), lambda i: (0, 0)),
        ],
        out_specs=pl.BlockSpec((_BLK, D_OUT), lambda i: (i, 0)),
        out_shape=jax.ShapeDtypeStruct((N, D_OUT), _f32),
    )(g, d, b2)


def kernel(features, edge_index, W1, b1, W2, b2):
    src_t = edge_index[0].reshape(NS, CH1, K)
    dst_t = edge_index[1].reshape(NS, CH1, K)
    src2 = jnp.stack([2 * src_t, 2 * src_t + 1])
    feat2 = features.reshape(2 * N, D_HALF)

    z_half = jnp.zeros((ROWS_PER_TILE, D_HALF), _f32)
    z_deg = jnp.zeros((ROWS_PER_TILE, 8), _f32)
    z_out = jnp.zeros((ROWS_PER_TILE, D_OUT), _f32)
    ones_rows = jnp.ones((K, 8), _f32)

    agg1, deg16 = _sc_agg1(feat2, src2, dst_t, z_half, z_deg, ones_rows)

    h2 = _tc_mid(agg1, deg16, W1, b1.reshape(1, D_HID), W2)

    agg2 = _sc_agg2(h2, src_t, dst_t, z_out)
    out = _tc_final(agg2, deg16, b2.reshape(1, D_OUT))
    return out
